# baseline (speedup 1.0000x reference)
import jax
import jax.numpy as jnp
import numpy as np
from jax.experimental import pallas as pl

_CHO_LEN = 19
_JUNG_LEN = 21
_JONG_LEN = 27
_SPECIAL = (' ', '"', "'", '(', ')', ',', '.', '?', '0', '1', '2', '3',
            '4', '5', '6', '7', '8', '9')
_cases = [chr(10)] + list(_SPECIAL)
_TBL = np.full(len(_SPECIAL) + 2, -1, dtype=np.int32)
_TBL[:len(_cases)] = np.asarray([ord(c) for c in _cases], dtype=np.int32)
_GA = 44032

_OFF_CHO = 1
_N_CHO = _CHO_LEN + 1
_OFF_JUNG = _OFF_CHO + _N_CHO
_N_JUNG = _JUNG_LEN + 1
_OFF_JONG = _OFF_JUNG + _N_JUNG
_N_JONG = _JONG_LEN + 1
_OFF_SPEC = _OFF_JONG + _N_JONG
_N_SPEC = len(_SPECIAL) + 2


def _argmax_lastdim(x):
    m = jnp.max(x, axis=-1, keepdims=True)
    idx = jax.lax.broadcasted_iota(jnp.int32, x.shape, x.ndim - 1)
    big = jnp.int32(x.shape[-1])
    cand = jnp.where(x >= m, idx, big)
    return jnp.min(cand, axis=-1)


def _body(x_ref, o_ref):
    x = x_ref[...]
    han_mask = x[:, :, 0] >= 0.5
    cho = _argmax_lastdim(x[:, :, _OFF_CHO:_OFF_CHO + _N_CHO])
    jung = _argmax_lastdim(x[:, :, _OFF_JUNG:_OFF_JUNG + _N_JUNG])
    jong = _argmax_lastdim(x[:, :, _OFF_JONG:_OFF_JONG + _N_JONG])
    spec = _argmax_lastdim(x[:, :, _OFF_SPEC:_OFF_SPEC + _N_SPEC])
    han_uni = (cho * _JUNG_LEN + jung) * _JONG_LEN + jong + _GA
    spec_uni = jnp.full_like(spec, -1)
    for k in range(_N_SPEC - 1):
        spec_uni = jnp.where(spec == k, jnp.int32(int(_TBL[k])), spec_uni)
    o_ref[...] = jnp.where(han_mask, han_uni, spec_uni)


def kernel(inputs):
    B, L, D = inputs.shape
    bb = 64
    grid = (B // bb,)
    return pl.pallas_call(
        _body,
        grid=grid,
        in_specs=[pl.BlockSpec((bb, L, D), lambda i: (i, 0, 0))],
        out_specs=pl.BlockSpec((bb, L), lambda i: (i, 0)),
        out_shape=jax.ShapeDtypeStruct((B, L), jnp.int32),
    )(inputs)


# speedup vs baseline: 3.6979x; 3.6979x over previous
import jax
import jax.numpy as jnp
import numpy as np
from jax import lax
from jax.experimental import pallas as pl
from jax.experimental.pallas import tpu as pltpu

_CHO_LEN = 19
_JUNG_LEN = 21
_JONG_LEN = 27
_SPECIAL = (' ', '"', "'", '(', ')', ',', '.', '?', '0', '1', '2', '3',
            '4', '5', '6', '7', '8', '9')
_cases = [chr(10)] + list(_SPECIAL)
_TBL = np.full(len(_SPECIAL) + 2, -1, dtype=np.int32)
_TBL[:len(_cases)] = np.asarray([ord(c) for c in _cases], dtype=np.int32)
_GA = 44032

_SEGS = ((1, 20), (21, 22), (43, 28), (71, 20))
_D = 91


def _body(x_ref, o_ref):
    xt = x_ref[...].T
    r = xt.shape[1]

    def seg_argmax(a, n):
        seg = lax.slice(xt, (a, 0), (a + n, r))
        m = jnp.max(seg, axis=0)
        iota = lax.broadcasted_iota(jnp.int32, (n, r), 0)
        cand = jnp.where(seg == m[None, :], iota, jnp.int32(n))
        return jnp.min(cand, axis=0)

    han = xt[0, :] >= 0.5
    cho = seg_argmax(*_SEGS[0])
    jung = seg_argmax(*_SEGS[1])
    jong = seg_argmax(*_SEGS[2])
    spec = seg_argmax(*_SEGS[3])
    han_uni = (cho * _JUNG_LEN + jung) * _JONG_LEN + jong + _GA
    spec_uni = jnp.full_like(spec, -1)
    for k in range(7, -1, -1):
        spec_uni = jnp.where(spec == k, jnp.int32(int(_TBL[k])), spec_uni)
    spec_uni = jnp.where((spec >= 8) & (spec <= 17), spec + 40, spec_uni)
    o_ref[...] = jnp.where(han, han_uni, spec_uni)


def kernel(inputs):
    B, L, D = inputs.shape
    n_rows = B * L
    x2 = inputs.reshape(n_rows, D)
    r = 2048
    grid = (n_rows // r,)
    out = pl.pallas_call(
        _body,
        grid=grid,
        in_specs=[pl.BlockSpec((r, D), lambda i: (i, 0))],
        out_specs=pl.BlockSpec((r,), lambda i: (i,)),
        out_shape=jax.ShapeDtypeStruct((n_rows,), jnp.int32),
        compiler_params=pltpu.CompilerParams(
            dimension_semantics=("parallel",)),
    )(x2)
    return out.reshape(B, L)


# speedup vs baseline: 4.8504x; 1.3117x over previous
import jax
import jax.numpy as jnp
import numpy as np
from jax import lax
from jax.experimental import pallas as pl
from jax.experimental.pallas import tpu as pltpu

_CHO_LEN = 19
_JUNG_LEN = 21
_JONG_LEN = 27
_SPECIAL = (' ', '"', "'", '(', ')', ',', '.', '?', '0', '1', '2', '3',
            '4', '5', '6', '7', '8', '9')
_cases = [chr(10)] + list(_SPECIAL)
_TBL = np.full(len(_SPECIAL) + 2, -1, dtype=np.int32)
_TBL[:len(_cases)] = np.asarray([ord(c) for c in _cases], dtype=np.int32)
_GA = 44032

_SEGS = ((1, 20), (21, 22), (43, 28), (71, 20))
_D = 91


def _body(x_ref, o_ref):
    xt = x_ref[...].T
    r = xt.shape[1]

    def seg_argmax(a, n):
        seg = lax.slice(xt, (a, 0), (a + n, r))
        m = jnp.max(seg, axis=0)
        iota = lax.broadcasted_iota(jnp.int32, (n, r), 0)
        cand = jnp.where(seg == m[None, :], iota, jnp.int32(n))
        return jnp.min(cand, axis=0)

    han = xt[0, :] >= 0.5
    cho = seg_argmax(*_SEGS[0])
    jung = seg_argmax(*_SEGS[1])
    jong = seg_argmax(*_SEGS[2])
    spec = seg_argmax(*_SEGS[3])
    han_uni = (cho * _JUNG_LEN + jung) * _JONG_LEN + jong + _GA
    spec_uni = jnp.full_like(spec, -1)
    for k in range(7, -1, -1):
        spec_uni = jnp.where(spec == k, jnp.int32(int(_TBL[k])), spec_uni)
    spec_uni = jnp.where((spec >= 8) & (spec <= 17), spec + 40, spec_uni)
    o_ref[...] = jnp.where(han, han_uni, spec_uni)


def kernel(inputs):
    B, L, D = inputs.shape
    n_rows = B * L
    x2 = inputs.reshape(n_rows, D)
    r = 8192
    grid = (n_rows // r,)
    out = pl.pallas_call(
        _body,
        grid=grid,
        in_specs=[pl.BlockSpec((r, D), lambda i: (i, 0))],
        out_specs=pl.BlockSpec((r,), lambda i: (i,)),
        out_shape=jax.ShapeDtypeStruct((n_rows,), jnp.int32),
        compiler_params=pltpu.CompilerParams(
            dimension_semantics=("parallel",)),
    )(x2)
    return out.reshape(B, L)
